# baseline (speedup 1.0000x reference)
import jax
import jax.numpy as jnp
from jax import lax
from jax.experimental import pallas as pl
from jax.experimental.pallas import tpu as pltpu
from jax.experimental.pallas import tpu_sc as plsc

N = 10000
E = 320000
D = 128
NP = 10240
NC = 2
NS = 16
NW = NC * NS
C = 128
K = 80
EP = NW * K * C
SEG = NP // NS

K0 = 80
K1 = 80
_f32 = jnp.float32


def _ZV():
    return jnp.zeros((16,), _f32)


def _ONEV():
    return jnp.ones((16,), _f32)

_mesh = plsc.VectorSubcoreMesh(
    core_axis_name="c", subcore_axis_name="s", num_cores=NC, num_subcores=NS)


def _zero_1d(ref, n):
    def body(i, _):
        ref[pl.ds(i * 16, 16)] = _ZV()
        return 0
    lax.fori_loop(0, n // 16, body, 0)


def _deg_body(e_hbm, out_hbm,
              ev, do_loc, di_loc, slots_sh, acc_v, tmp_v):
    c = lax.axis_index("c")
    s = lax.axis_index("s")
    w = c * NS + s
    pltpu.sync_copy(e_hbm.at[0, pl.ds(w * K * C, K * C)], ev.at[0])
    pltpu.sync_copy(e_hbm.at[1, pl.ds(w * K * C, K * C)], ev.at[1])
    _zero_1d(do_loc, NP)
    _zero_1d(di_loc, NP)

    def hloop(j, _):
        one = _ONEV()
        for i in range(C // 16):
            si = ev[0, pl.ds(j * C + i * 16, 16)]
            di = ev[1, pl.ds(j * C + i * 16, 16)]
            plsc.addupdate_scatter(do_loc, [si], one)
            plsc.addupdate_scatter(di_loc, [di], one)
        return 0
    lax.fori_loop(0, K, hloop, 0)

    for r, loc in ((0, do_loc), (1, di_loc)):
        pltpu.sync_copy(loc, slots_sh.at[s])
        plsc.subcore_barrier()
        _zero_1d(acc_v, SEG)

        def rloop(k, _):
            pltpu.sync_copy(slots_sh.at[k, pl.ds(s * SEG, SEG)], tmp_v)

            def aloop(i, _):
                acc_v[pl.ds(i * 16, 16)] = (acc_v[pl.ds(i * 16, 16)]
                                            + tmp_v[pl.ds(i * 16, 16)])
                return 0
            lax.fori_loop(0, SEG // 16, aloop, 0)
            return 0
        lax.fori_loop(0, NS, rloop, 0)
        pltpu.sync_copy(acc_v, out_hbm.at[c, r, pl.ds(s * SEG, SEG)])
        plsc.subcore_barrier()


_deg_call = pl.kernel(
    _deg_body,
    out_type=jax.ShapeDtypeStruct((NC, 2, NP), _f32),
    mesh=_mesh,
    scratch_types=[
        pltpu.VMEM((2, K * C), jnp.int32),
        pltpu.VMEM((NP,), _f32),
        pltpu.VMEM((NP,), _f32),
        pltpu.VMEM_SHARED((NS, NP), _f32),
        pltpu.VMEM((SEG,), _f32),
        pltpu.VMEM((SEG,), _f32),
    ],
    compiler_params=pltpu.CompilerParams(needs_layout_passes=False),
)


def _agg_body(e_hbm, h_hbm, nd_hbm, agg_out, s_out,
              eb, r0, r1, nd0, nd1, nd2, nd3,
              zrow_v, zs_v, agg_sh, s_sh,
              g0, g1, sc0, sc1, n0, n1, n2, n3, t0, t1, t2, t3,
              i0, i1, i2, i3, i4, i5, i6, i7):
    c = lax.axis_index("c")
    s = lax.axis_index("s")
    w = c * NS + s
    rows = (r0, r1)
    ndv = (nd0, nd1, nd2, nd3)
    gsem = (g0, g1)
    ssem = (sc0, sc1)
    nsem = (n0, n1, n2, n3)
    s2sem = (t0, t1, t2, t3)
    isem = (i0, i1, i2, i3, i4, i5, i6, i7)
    ch0 = w * K
    NT = K // 8

    def idx_load(j, b):
        pltpu.async_copy(e_hbm.at[0, pl.ds((ch0 + j) * C, C)], eb.at[b, 0],
                         isem[b])
        pltpu.async_copy(e_hbm.at[1, pl.ds((ch0 + j) * C, C)], eb.at[b, 1],
                         isem[b])

    def idx_wait(b):
        pltpu.make_async_copy(e_hbm.at[pl.ds(0, 2), pl.ds(0, C)], eb.at[b],
                              isem[b]).wait()

    h_c = h_hbm.at[c]
    for b in range(5):
        idx_load(b, b)
    idx_wait(0)
    idx_wait(1)
    pltpu.async_copy(h_c.at[eb.at[0, 0]], rows[0], gsem[0])
    pltpu.async_copy(nd_hbm.at[eb.at[0, 1]], ndv[0], nsem[0])
    pltpu.async_copy(nd_hbm.at[eb.at[1, 1]], ndv[1], nsem[1])

    def zr(i, _):
        for t in range(8):
            zrow_v[i, pl.ds(t * 16, 16)] = _ZV()
        return 0
    lax.fori_loop(0, 8, zr, 0)
    _zero_1d(zs_v, SEG)

    def zagg(i, _):
        pltpu.sync_copy(zrow_v, agg_sh.at[pl.ds(s * SEG + i * 8, 8)])
        return 0
    lax.fori_loop(0, SEG // 8, zagg, 0)
    pltpu.sync_copy(zs_v, s_sh.at[pl.ds(s * SEG, SEG)])
    plsc.subcore_barrier()

    def ring(tt, _):
        for u in range(8):
            j = 8 * tt + u
            rb = u % 2
            ro = 1 - rb
            nb = u % 4
            nbn = (u + 2) % 4
            srcb = eb.at[u, 0]
            dstb = eb.at[u, 1]
            def issue_self():
                pltpu.async_copy(h_c.at[srcb], rows[rb], gsem[rb])
            if u == 0:
                pl.when(tt >= 1)(issue_self)
            else:
                issue_self()
            pltpu.make_async_copy(h_c.at[srcb], rows[rb], gsem[rb]).wait()
            pltpu.sync_copy(rows[rb], agg_sh.at[dstb], add=True)

            pltpu.make_async_copy(nd_hbm.at[dstb], ndv[nb], nsem[nb]).wait()
            pltpu.async_copy(ndv[nb], s_sh.at[srcb], s2sem[nb], add=True)

            def wait_s2():
                pltpu.make_async_copy(ndv[nbn], s_sh.at[srcb],
                                      s2sem[nbn]).wait()

            def issue_nd():
                idx_wait((u + 2) % 8)
                pltpu.async_copy(nd_hbm.at[eb.at[(u + 2) % 8, 1]], ndv[nbn],
                                 nsem[nbn])
            if u < 2:
                pl.when(tt >= 1)(wait_s2)
            else:
                wait_s2()
            if u >= 6:
                pl.when(tt <= NT - 2)(issue_nd)
            else:
                issue_nd()

            def issue_idx():
                idx_load(j + 5, (u + 5) % 8)
            if u >= 3:
                pl.when(tt <= NT - 2)(issue_idx)
            else:
                issue_idx()
        return 0
    lax.fori_loop(0, NT, ring, 0)
    pltpu.make_async_copy(ndv[2], s_sh.at[eb.at[0, 0]], s2sem[2]).wait()
    pltpu.make_async_copy(ndv[3], s_sh.at[eb.at[0, 0]], s2sem[3]).wait()
    plsc.subcore_barrier()

    pltpu.sync_copy(agg_sh.at[pl.ds(s * SEG, SEG)],
                    agg_out.at[c, pl.ds(s * SEG, SEG)])
    pltpu.sync_copy(s_sh.at[pl.ds(s * SEG, SEG)],
                    s_out.at[c, pl.ds(s * SEG, SEG)])


_agg_call = pl.kernel(
    _agg_body,
    out_type=(jax.ShapeDtypeStruct((NC, NP, D), _f32),
              jax.ShapeDtypeStruct((NC, NP), _f32)),
    mesh=_mesh,
    scratch_types=(
        [pltpu.VMEM((8, 2, C), jnp.int32),
         pltpu.VMEM((C, D), _f32),
         pltpu.VMEM((C, D), _f32)]
        + [pltpu.VMEM((C,), _f32) for _ in range(4)]
        + [pltpu.VMEM((8, D), _f32),
           pltpu.VMEM((SEG,), _f32),
           pltpu.VMEM_SHARED((NP, D), _f32),
           pltpu.VMEM_SHARED((NP,), _f32)]
        + [pltpu.SemaphoreType.DMA for _ in range(20)]
    ),
    compiler_params=pltpu.CompilerParams(needs_layout_passes=False),
)


def _pad_body(e_ref, out_ref):
    out_ref[:, 0:E] = e_ref[...]
    out_ref[:, E:EP] = jnp.full((2, EP - E), N, jnp.int32)


_pad_call = pl.pallas_call(
    _pad_body,
    out_shape=jax.ShapeDtypeStruct((2, EP), jnp.int32),
)


def _k1_body(dp_ref, x_ref, w1_ref, h_ref, nsnd_ref):
    dp = dp_ref[...]
    deg_o = dp[:, 0:1] + dp[:, 2:3]
    deg_i = dp[:, 1:2] + dp[:, 3:4]
    ns = jnp.where(deg_o > 0, lax.rsqrt(jnp.maximum(deg_o, 1e-12)), 0.0)
    nd = jnp.where(deg_i > 0, lax.rsqrt(jnp.maximum(deg_i, 1e-12)), 0.0)
    nsnd_ref[...] = jnp.concatenate([ns, nd], axis=1)
    x = x_ref[...]
    h = jnp.dot(x * ns, w1_ref[...], preferred_element_type=jnp.float32)
    h_ref[0] = h
    h_ref[1] = h


_k1_call = pl.pallas_call(
    _k1_body,
    out_shape=(jax.ShapeDtypeStruct((2, NP, D), _f32),
               jax.ShapeDtypeStruct((NP, 2), _f32)),
)


def _k2_body(ap_ref, sp_ref, nsnd_ref, b1_ref, w2_ref, b2_ref, wc_ref,
             bc_ref, out_ref):
    agg = ap_ref[0] + ap_ref[1]
    sp = sp_ref[...]
    srow = sp[:, 0:1] + sp[:, 1:2]
    ns = nsnd_ref[:, 0:1]
    nd = nsnd_ref[:, 1:2]
    h1 = jnp.maximum(agg * nd + b1_ref[...], 0.0)
    rowmask = lax.broadcasted_iota(jnp.int32, (NP, 1), 0) < N
    cvec = jnp.where(rowmask, ns * srow, 0.0)
    v = jnp.sum(h1 * cvec, axis=0, keepdims=True)
    hg = (jnp.dot(v, w2_ref[...], preferred_element_type=jnp.float32)
          * (1.0 / N) + b2_ref[...])
    out_ref[...] = (jnp.dot(hg, wc_ref[...],
                            preferred_element_type=jnp.float32) + bc_ref[...])


_k2_call = pl.pallas_call(
    _k2_body,
    out_shape=jax.ShapeDtypeStruct((1, 2), _f32),
)


def kernel(features, edge_index, W1, b1, W2, b2, Wc, bc):
    edges = _pad_call(edge_index)

    deg = _deg_call(edges)
    dp4 = deg.reshape(4, NP).T
    xpad = jnp.concatenate(
        [features, jnp.zeros((NP - N, D), _f32)], axis=0)
    H, nsnd = _k1_call(dp4, xpad, W1)
    nd_flat = nsnd[:, 1]

    aggp, sp = _agg_call(edges, H, nd_flat)
    sp2 = sp.reshape(NC, NP).T
    logits = _k2_call(aggp, sp2, nsnd,
                      b1.reshape(1, D), W2, b2.reshape(1, D),
                      Wc, bc.reshape(1, 2))
    return logits


# speedup vs baseline: 1.1521x; 1.1521x over previous
import jax
import jax.numpy as jnp
from jax import lax
from jax.experimental import pallas as pl
from jax.experimental.pallas import tpu as pltpu
from jax.experimental.pallas import tpu_sc as plsc

N = 10000
E = 320000
D = 128
NP = 10240
NC = 2
NS = 16
NW = NC * NS
C = 128
K = -(-E // (NW * C))
EP = NW * K * C
SEG = NP // NS

_f32 = jnp.float32


def _ZV():
    return jnp.zeros((16,), _f32)


def _ONEV():
    return jnp.ones((16,), _f32)


_mesh = plsc.VectorSubcoreMesh(
    core_axis_name="c", subcore_axis_name="s", num_cores=NC, num_subcores=NS)


def _zero_1d(ref, n):
    def body(i, _):
        ref[pl.ds(i * 16, 16)] = _ZV()
        return 0
    lax.fori_loop(0, n // 16, body, 0)


def _deg_body(src_hbm, dst_hbm, out_hbm,
              src_v, dst_v, do_loc, di_loc, slots_sh, acc_v, tmp_v):
    c = lax.axis_index("c")
    s = lax.axis_index("s")
    w = c * NS + s
    pltpu.sync_copy(src_hbm.at[w], src_v)
    pltpu.sync_copy(dst_hbm.at[w], dst_v)
    _zero_1d(do_loc, NP)
    _zero_1d(di_loc, NP)

    def hloop(j, _):
        one = _ONEV()
        for i in range(C // 16):
            si = src_v[j, pl.ds(i * 16, 16)]
            di = dst_v[j, pl.ds(i * 16, 16)]
            plsc.addupdate_scatter(do_loc, [si], one)
            plsc.addupdate_scatter(di_loc, [di], one)
        return 0
    lax.fori_loop(0, K, hloop, 0)

    for r, loc in ((0, do_loc), (1, di_loc)):
        pltpu.sync_copy(loc, slots_sh.at[s])
        plsc.subcore_barrier()
        _zero_1d(acc_v, SEG)

        def rloop(k, _):
            pltpu.sync_copy(slots_sh.at[k, pl.ds(s * SEG, SEG)], tmp_v)

            def aloop(i, _):
                acc_v[pl.ds(i * 16, 16)] = (acc_v[pl.ds(i * 16, 16)]
                                            + tmp_v[pl.ds(i * 16, 16)])
                return 0
            lax.fori_loop(0, SEG // 16, aloop, 0)
            return 0
        lax.fori_loop(0, NS, rloop, 0)
        pltpu.sync_copy(acc_v, out_hbm.at[c, r, pl.ds(s * SEG, SEG)])
        plsc.subcore_barrier()


_deg_call = pl.kernel(
    _deg_body,
    out_type=jax.ShapeDtypeStruct((NC, 2, NP), _f32),
    mesh=_mesh,
    scratch_types=[
        pltpu.VMEM((K, C), jnp.int32),
        pltpu.VMEM((K, C), jnp.int32),
        pltpu.VMEM((NP,), _f32),
        pltpu.VMEM((NP,), _f32),
        pltpu.VMEM_SHARED((NS, NP), _f32),
        pltpu.VMEM((SEG,), _f32),
        pltpu.VMEM((SEG,), _f32),
    ],
    compiler_params=pltpu.CompilerParams(needs_layout_passes=False),
)


def _agg_body(src_hbm, dst_hbm, h_hbm, nd_hbm, agg_out, s_out,
              src_v, dst_v, rows_v, ndv, zrow_v, zs_v, agg_sh, s_sh, sem):
    c = lax.axis_index("c")
    s = lax.axis_index("s")
    w = c * NS + s
    pltpu.sync_copy(src_hbm.at[w], src_v)
    pltpu.sync_copy(dst_hbm.at[w], dst_v)

    def zr(i, _):
        for t in range(8):
            zrow_v[i, pl.ds(t * 16, 16)] = _ZV()
        return 0
    lax.fori_loop(0, 16, zr, 0)
    _zero_1d(zs_v, SEG)

    def zagg(i, _):
        pltpu.sync_copy(zrow_v, agg_sh.at[pl.ds(s * SEG + i * 16, 16)])
        return 0
    lax.fori_loop(0, SEG // 16, zagg, 0)
    pltpu.sync_copy(zs_v, s_sh.at[pl.ds(s * SEG, SEG)])
    plsc.subcore_barrier()

    def chunk(j, _):
        pltpu.async_copy(h_hbm.at[src_v.at[j]], rows_v, sem).wait()
        pltpu.sync_copy(rows_v, agg_sh.at[dst_v.at[j]], add=True)
        pltpu.async_copy(nd_hbm.at[dst_v.at[j]], ndv, sem).wait()
        pltpu.sync_copy(ndv, s_sh.at[src_v.at[j]], add=True)
        return 0
    lax.fori_loop(0, K, chunk, 0)
    plsc.subcore_barrier()

    pltpu.sync_copy(agg_sh.at[pl.ds(s * SEG, SEG)],
                    agg_out.at[c, pl.ds(s * SEG, SEG)])
    pltpu.sync_copy(s_sh.at[pl.ds(s * SEG, SEG)],
                    s_out.at[c, pl.ds(s * SEG, SEG)])


_agg_call = pl.kernel(
    _agg_body,
    out_type=(jax.ShapeDtypeStruct((NC, NP, D), _f32),
              jax.ShapeDtypeStruct((NC, NP), _f32)),
    mesh=_mesh,
    scratch_types=[
        pltpu.VMEM((K, C), jnp.int32),
        pltpu.VMEM((K, C), jnp.int32),
        pltpu.VMEM((C, D), _f32),
        pltpu.VMEM((C,), _f32),
        pltpu.VMEM((16, D), _f32),
        pltpu.VMEM((SEG,), _f32),
        pltpu.VMEM_SHARED((NP, D), _f32),
        pltpu.VMEM_SHARED((NP,), _f32),
        pltpu.SemaphoreType.DMA,
    ],
    compiler_params=pltpu.CompilerParams(needs_layout_passes=False),
)


def _k1_body(dp_ref, x_ref, w1_ref, h_ref, nsnd_ref):
    dp = dp_ref[...]
    deg_o = dp[:, 0:1] + dp[:, 2:3]
    deg_i = dp[:, 1:2] + dp[:, 3:4]
    ns = jnp.where(deg_o > 0, lax.rsqrt(jnp.maximum(deg_o, 1e-12)), 0.0)
    nd = jnp.where(deg_i > 0, lax.rsqrt(jnp.maximum(deg_i, 1e-12)), 0.0)
    nsnd_ref[...] = jnp.concatenate([ns, nd], axis=1)
    x = x_ref[...]
    h_ref[...] = jnp.dot(x * ns, w1_ref[...],
                         preferred_element_type=jnp.float32)


_k1_call = pl.pallas_call(
    _k1_body,
    out_shape=(jax.ShapeDtypeStruct((NP, D), _f32),
               jax.ShapeDtypeStruct((NP, 2), _f32)),
)


def _k2_body(ap_ref, sp_ref, nsnd_ref, b1_ref, w2_ref, b2_ref, wc_ref,
             bc_ref, out_ref):
    agg = ap_ref[0] + ap_ref[1]
    sp = sp_ref[...]
    srow = sp[:, 0:1] + sp[:, 1:2]
    ns = nsnd_ref[:, 0:1]
    nd = nsnd_ref[:, 1:2]
    h1 = jnp.maximum(agg * nd + b1_ref[...], 0.0)
    rowmask = lax.broadcasted_iota(jnp.int32, (NP, 1), 0) < N
    cvec = jnp.where(rowmask, ns * srow, 0.0)
    v = jnp.sum(h1 * cvec, axis=0, keepdims=True)
    hg = (jnp.dot(v, w2_ref[...], preferred_element_type=jnp.float32)
          * (1.0 / N) + b2_ref[...])
    out_ref[...] = (jnp.dot(hg, wc_ref[...],
                            preferred_element_type=jnp.float32) + bc_ref[...])


_k2_call = pl.pallas_call(
    _k2_body,
    out_shape=jax.ShapeDtypeStruct((1, 2), _f32),
)


def kernel(features, edge_index, W1, b1, W2, b2, Wc, bc):
    pad = jnp.full((EP - E,), N, jnp.int32)
    src = jnp.concatenate([edge_index[0], pad]).reshape(NW, K, C)
    dst = jnp.concatenate([edge_index[1], pad]).reshape(NW, K, C)

    deg = _deg_call(src, dst)
    dp4 = deg.reshape(4, NP).T
    xpad = jnp.concatenate(
        [features, jnp.zeros((NP - N, D), _f32)], axis=0)
    H, nsnd = _k1_call(dp4, xpad, W1)
    nd_flat = nsnd[:, 1]

    aggp, sp = _agg_call(src, dst, H, nd_flat)
    sp2 = sp.reshape(NC, NP).T
    logits = _k2_call(aggp, sp2, nsnd,
                      b1.reshape(1, D), W2, b2.reshape(1, D),
                      Wc, bc.reshape(1, 2))
    return logits
